# baseline (speedup 1.0000x reference)
import functools

import jax
import jax.numpy as jnp
from jax import lax
from jax.experimental import pallas as pl
from jax.experimental.pallas import tpu as pltpu
from jax.experimental.pallas import tpu_sc as plsc

N = 10000
E = 320000
D = 128

NC = 2
NS = 16
NW = NC * NS

EDGES_PER_WORKER = E // NW
BLK = 80
NBLK = EDGES_PER_WORKER // BLK
RCHUNK = 80
NRCHUNK = N // RCHUNK
ROUNDS = (NRCHUNK + NS - 1) // NS
RI = 6
RR = 4


def _sc_aggregate(ndata, eidx_flat, zrows, zdeg, ones_blk):
    mesh = plsc.VectorSubcoreMesh(core_axis_name="c", subcore_axis_name="s")

    @functools.partial(
        pl.kernel,
        mesh=mesh,
        out_type=(
            jax.ShapeDtypeStruct((NC, N, D), jnp.float32),
            jax.ShapeDtypeStruct((N,), jnp.float32),
            jax.ShapeDtypeStruct((N,), jnp.float32),
        ),
        scratch_types=[
            pltpu.VMEM((RI, BLK), jnp.int32),
            pltpu.VMEM((RI, BLK), jnp.int32),
            pltpu.VMEM((RR, BLK, D), jnp.float32),
            pltpu.VMEM((RCHUNK,), jnp.float32),
            pltpu.VMEM((BLK,), jnp.float32),
            pltpu.VMEM_SHARED((N, D), jnp.float32),
            pltpu.VMEM_SHARED((N,), jnp.float32),
            pltpu.SemaphoreType.DMA((RI,)),
            pltpu.SemaphoreType.DMA((RR,)),
            pltpu.SemaphoreType.DMA((2,)),
        ],
    )
    def k(ndata_hbm, eidx_hbm, zrows_hbm, zdeg_hbm, ones_hbm,
          acc_out, deg0_out, deg1_out, sidx, didx, rows, dstage,
          ones_v, acc, dacc, semi, semg, sems):
        c = lax.axis_index("c")
        s = lax.axis_index("s")

        pltpu.sync_copy(zrows_hbm, rows.at[0])
        pltpu.sync_copy(zdeg_hbm, dstage)
        pltpu.sync_copy(ones_hbm, ones_v)
        for kk in range(ROUNDS):
            cid = s + NS * kk

            @pl.when(cid < NRCHUNK)
            def _():
                r0 = cid * RCHUNK
                pltpu.sync_copy(rows.at[0], acc.at[pl.ds(r0, RCHUNK)])
                pltpu.sync_copy(dstage, dacc.at[pl.ds(r0, RCHUNK)])

        plsc.subcore_barrier()

        chunk_base = (c * NS + s) * EDGES_PER_WORKER

        def fire_idx(blk):
            isl = lax.rem(blk, RI)
            base = chunk_base + blk * BLK
            pltpu.async_copy(eidx_hbm.at[pl.ds(base, BLK)], sidx.at[isl],
                             semi.at[isl])
            pltpu.async_copy(eidx_hbm.at[pl.ds(E + base, BLK)],
                             didx.at[isl], semi.at[isl])

        def wait_idx(blk):
            isl = lax.rem(blk, RI)
            base = chunk_base + blk * BLK
            pltpu.make_async_copy(eidx_hbm.at[pl.ds(base, BLK)],
                                  sidx.at[isl], semi.at[isl]).wait()
            pltpu.make_async_copy(eidx_hbm.at[pl.ds(E + base, BLK)],
                                  didx.at[isl], semi.at[isl]).wait()

        def fire_gather(blk):
            isl = lax.rem(blk, RI)
            rsl = lax.rem(blk, RR)
            pltpu.async_copy(ndata_hbm.at[sidx.at[isl]], rows.at[rsl],
                             semg.at[rsl])

        def wait_gather(blk):
            isl = lax.rem(blk, RI)
            rsl = lax.rem(blk, RR)
            pltpu.make_async_copy(ndata_hbm.at[sidx.at[isl]], rows.at[rsl],
                                  semg.at[rsl]).wait()

        def fire_scat(blk):
            isl = lax.rem(blk, RI)
            rsl = lax.rem(blk, RR)
            ssl = lax.rem(blk, 2)
            pltpu.async_copy(rows.at[rsl], acc.at[didx.at[isl]],
                             sems.at[ssl], add=True)
            pltpu.async_copy(ones_v, dacc.at[didx.at[isl]], sems.at[ssl],
                             add=True)

        def wait_scat(blk):
            isl = lax.rem(blk, RI)
            rsl = lax.rem(blk, RR)
            ssl = lax.rem(blk, 2)
            pltpu.make_async_copy(rows.at[rsl], acc.at[didx.at[isl]],
                                  sems.at[ssl]).wait()
            pltpu.make_async_copy(ones_v, dacc.at[didx.at[isl]],
                                  sems.at[ssl]).wait()

        fire_idx(0)
        fire_idx(1)
        fire_idx(2)
        fire_idx(3)
        wait_idx(0)
        fire_gather(0)
        wait_idx(1)
        fire_gather(1)

        def body(i, carry):
            @pl.when(i >= 2)
            def _():
                wait_scat(i - 2)

            @pl.when(i + 4 < NBLK)
            def _():
                fire_idx(i + 4)

            @pl.when(i + 2 < NBLK)
            def _():
                wait_idx(i + 2)
                fire_gather(i + 2)

            wait_gather(i)
            fire_scat(i)
            return carry

        lax.fori_loop(0, NBLK, body, 0)
        wait_scat(NBLK - 2)
        wait_scat(NBLK - 1)
        plsc.subcore_barrier()

        for kk in range(ROUNDS):
            cid = s + NS * kk

            @pl.when(cid < NRCHUNK)
            def _():
                r0 = cid * RCHUNK
                pltpu.sync_copy(acc.at[pl.ds(r0, RCHUNK)],
                                acc_out.at[c, pl.ds(r0, RCHUNK)])
                pltpu.sync_copy(dacc.at[pl.ds(r0, RCHUNK)], dstage)

                @pl.when(c == 0)
                def _():
                    pltpu.sync_copy(dstage, deg0_out.at[pl.ds(r0, RCHUNK)])

                @pl.when(c == 1)
                def _():
                    pltpu.sync_copy(dstage, deg1_out.at[pl.ds(r0, RCHUNK)])

    return k(ndata, eidx_flat, zrows, zdeg, ones_blk)


def _sc_finish(ndata_flat, acc_flat, deg0, deg1):
    mesh = plsc.VectorSubcoreMesh(core_axis_name="c", subcore_axis_name="s")
    NV = RCHUNK // 16
    CW = RCHUNK * D

    @functools.partial(
        pl.kernel,
        mesh=mesh,
        out_type=jax.ShapeDtypeStruct((N * D,), jnp.float32),
        scratch_types=[
            pltpu.VMEM((CW,), jnp.float32),
            pltpu.VMEM((CW,), jnp.float32),
            pltpu.VMEM((CW,), jnp.float32),
            pltpu.VMEM((CW,), jnp.float32),
            pltpu.VMEM((RCHUNK,), jnp.float32),
            pltpu.VMEM((RCHUNK,), jnp.float32),
            pltpu.VMEM((RCHUNK,), jnp.float32),
        ],
    )
    def k(nd_hbm, acc_hbm, d0_hbm, d1_hbm, rst_out,
          nd_v, p0_v, p1_v, o_v, d0_v, d1_v, r_v):
        c = lax.axis_index("c")
        s = lax.axis_index("s")
        w = c * NS + s

        def do_chunk(cid):
            r0 = cid * RCHUNK
            f0 = r0 * D
            pltpu.sync_copy(nd_hbm.at[pl.ds(f0, CW)], nd_v)
            pltpu.sync_copy(acc_hbm.at[pl.ds(f0, CW)], p0_v)
            pltpu.sync_copy(acc_hbm.at[pl.ds(N * D + f0, CW)], p1_v)
            pltpu.sync_copy(d0_hbm.at[pl.ds(r0, RCHUNK)], d0_v)
            pltpu.sync_copy(d1_hbm.at[pl.ds(r0, RCHUNK)], d1_v)
            for v in range(NV):
                sl = pl.ds(v * 16, 16)
                dsum = d0_v[sl] + d1_v[sl]
                r_v[sl] = 1.0 / jnp.maximum(dsum, 1.0)

            def row(j, carry):
                rv16 = r_v[pl.ds((j // 16) * 16, 16)]
                rb = lax.gather(
                    rv16, jnp.full((16, 1), j % 16, jnp.int32),
                    lax.GatherDimensionNumbers(
                        offset_dims=(), collapsed_slice_dims=(0,),
                        start_index_map=(0,)),
                    (1,), mode=lax.GatherScatterMode.PROMISE_IN_BOUNDS)
                for cc in range(D // 16):
                    csl = pl.ds(j * D + cc * 16, 16)
                    o_v[csl] = nd_v[csl] + (p0_v[csl] + p1_v[csl]) * rb
                return carry

            lax.fori_loop(0, RCHUNK, row, 0)
            pltpu.sync_copy(o_v, rst_out.at[pl.ds(f0, CW)])

        for kk in range(4):
            cid = w + NW * kk

            @pl.when(cid < NRCHUNK)
            def _():
                do_chunk(cid)

    return k(ndata_flat, acc_flat, deg0, deg1)


ROW_BLK = 1000


def _tc_linear_body(rst_ref, w_ref, b_ref, o_ref):
    o_ref[...] = (
        lax.dot_general(rst_ref[...], w_ref[...], (((1,), (1,)), ((), ())),
                        preferred_element_type=jnp.float32)
        + b_ref[...]
    )


def _tc_linear(rst, w, b2):
    grid = (N // ROW_BLK,)
    return pl.pallas_call(
        _tc_linear_body,
        grid=grid,
        in_specs=[
            pl.BlockSpec((ROW_BLK, D), lambda i: (i, 0)),
            pl.BlockSpec((D, D), lambda i: (0, 0)),
            pl.BlockSpec((1, D), lambda i: (0, 0)),
        ],
        out_specs=pl.BlockSpec((ROW_BLK, D), lambda i: (i, 0)),
        out_shape=jax.ShapeDtypeStruct((N, D), jnp.float32),
    )(rst, w, b2)


@jax.jit
def kernel(ndata, edge_index, W, b):
    eidx_flat = edge_index.reshape(2 * E)
    zrows = jnp.zeros((RCHUNK, D), jnp.float32)
    zdeg = jnp.zeros((RCHUNK,), jnp.float32)
    ones_blk = jnp.ones((BLK,), jnp.float32)

    acc, deg0, deg1 = _sc_aggregate(ndata, eidx_flat, zrows, zdeg, ones_blk)
    rst = _sc_finish(ndata.reshape(N * D), acc.reshape(NC * N * D),
                     deg0, deg1)

    b2 = b.reshape(1, D)
    return _tc_linear(rst.reshape(N, D), W, b2)


# speedup vs baseline: 1.2717x; 1.2717x over previous
import functools

import jax
import jax.numpy as jnp
from jax import lax
from jax.experimental import pallas as pl
from jax.experimental.pallas import tpu as pltpu
from jax.experimental.pallas import tpu_sc as plsc

N = 10000
E = 320000
D = 128

NC = 2
NS = 16
NW = NC * NS

EDGES_PER_WORKER = E // NW
BLK = 80
NBLK = EDGES_PER_WORKER // BLK
RCHUNK = 80
NRCHUNK = N // RCHUNK
ROUNDS = (NRCHUNK + NS - 1) // NS
RI = 6
RR = 4


def _sc_aggregate(ndata, eidx_flat, zrows, zdeg, ones_blk):
    mesh = plsc.VectorSubcoreMesh(core_axis_name="c", subcore_axis_name="s")

    @functools.partial(
        pl.kernel,
        mesh=mesh,
        out_type=(
            jax.ShapeDtypeStruct((NC, N, D), jnp.float32),
            jax.ShapeDtypeStruct((N,), jnp.float32),
            jax.ShapeDtypeStruct((N,), jnp.float32),
        ),
        scratch_types=[
            pltpu.VMEM((RI, BLK), jnp.int32),
            pltpu.VMEM((RI, BLK), jnp.int32),
            pltpu.VMEM((RR, BLK, D), jnp.float32),
            pltpu.VMEM((ROUNDS, RCHUNK), jnp.float32),
            pltpu.VMEM((BLK,), jnp.float32),
            pltpu.VMEM_SHARED((N, D), jnp.float32),
            pltpu.VMEM_SHARED((N,), jnp.float32),
            pltpu.SemaphoreType.DMA((RI,)),
            pltpu.SemaphoreType.DMA((RR,)),
            pltpu.SemaphoreType.DMA((2,)),
            pltpu.SemaphoreType.DMA,
        ],
    )
    def k(ndata_hbm, eidx_hbm, zrows_hbm, zdeg_hbm, ones_hbm,
          acc_out, deg0_out, deg1_out, sidx, didx, rows, dstage,
          ones_v, acc, dacc, semi, semg, sems, semz):
        c = lax.axis_index("c")
        s = lax.axis_index("s")

        pltpu.async_copy(zrows_hbm, rows.at[0], semz)
        pltpu.async_copy(zdeg_hbm, dstage.at[0], semz)
        pltpu.async_copy(ones_hbm, ones_v, semz)
        pltpu.make_async_copy(zrows_hbm, rows.at[0], semz).wait()
        pltpu.make_async_copy(zdeg_hbm, dstage.at[0], semz).wait()
        pltpu.make_async_copy(ones_hbm, ones_v, semz).wait()
        for kk in range(ROUNDS):
            cid = s + NS * kk

            @pl.when(cid < NRCHUNK)
            def _():
                r0 = cid * RCHUNK
                pltpu.async_copy(rows.at[0], acc.at[pl.ds(r0, RCHUNK)],
                                 semz)
                pltpu.async_copy(dstage.at[0], dacc.at[pl.ds(r0, RCHUNK)],
                                 semz)

        for kk in range(ROUNDS):
            cid = s + NS * kk

            @pl.when(cid < NRCHUNK)
            def _():
                r0 = cid * RCHUNK
                pltpu.make_async_copy(rows.at[0],
                                      acc.at[pl.ds(r0, RCHUNK)],
                                      semz).wait()
                pltpu.make_async_copy(dstage.at[0],
                                      dacc.at[pl.ds(r0, RCHUNK)],
                                      semz).wait()

        plsc.subcore_barrier()

        chunk_base = (c * NS + s) * EDGES_PER_WORKER

        def fire_idx(blk):
            isl = lax.rem(blk, RI)
            base = chunk_base + blk * BLK
            pltpu.async_copy(eidx_hbm.at[pl.ds(base, BLK)], sidx.at[isl],
                             semi.at[isl])
            pltpu.async_copy(eidx_hbm.at[pl.ds(E + base, BLK)],
                             didx.at[isl], semi.at[isl])

        def wait_idx(blk):
            isl = lax.rem(blk, RI)
            base = chunk_base + blk * BLK
            pltpu.make_async_copy(eidx_hbm.at[pl.ds(base, BLK)],
                                  sidx.at[isl], semi.at[isl]).wait()
            pltpu.make_async_copy(eidx_hbm.at[pl.ds(E + base, BLK)],
                                  didx.at[isl], semi.at[isl]).wait()

        def fire_gather(blk):
            isl = lax.rem(blk, RI)
            rsl = lax.rem(blk, RR)
            pltpu.async_copy(ndata_hbm.at[sidx.at[isl]], rows.at[rsl],
                             semg.at[rsl])

        def wait_gather(blk):
            isl = lax.rem(blk, RI)
            rsl = lax.rem(blk, RR)
            pltpu.make_async_copy(ndata_hbm.at[sidx.at[isl]], rows.at[rsl],
                                  semg.at[rsl]).wait()

        def fire_scat(blk):
            isl = lax.rem(blk, RI)
            rsl = lax.rem(blk, RR)
            ssl = lax.rem(blk, 2)
            pltpu.async_copy(rows.at[rsl], acc.at[didx.at[isl]],
                             sems.at[ssl], add=True)
            pltpu.async_copy(ones_v, dacc.at[didx.at[isl]], sems.at[ssl],
                             add=True)

        def wait_scat(blk):
            isl = lax.rem(blk, RI)
            rsl = lax.rem(blk, RR)
            ssl = lax.rem(blk, 2)
            pltpu.make_async_copy(rows.at[rsl], acc.at[didx.at[isl]],
                                  sems.at[ssl]).wait()
            pltpu.make_async_copy(ones_v, dacc.at[didx.at[isl]],
                                  sems.at[ssl]).wait()

        fire_idx(0)
        fire_idx(1)
        fire_idx(2)
        fire_idx(3)
        wait_idx(0)
        fire_gather(0)
        wait_idx(1)
        fire_gather(1)

        def body(i, carry):
            @pl.when(i >= 2)
            def _():
                wait_scat(i - 2)

            @pl.when(i + 4 < NBLK)
            def _():
                fire_idx(i + 4)

            @pl.when(i + 2 < NBLK)
            def _():
                wait_idx(i + 2)
                fire_gather(i + 2)

            wait_gather(i)
            fire_scat(i)
            return carry

        lax.fori_loop(0, NBLK, body, 0)
        wait_scat(NBLK - 2)
        wait_scat(NBLK - 1)
        plsc.subcore_barrier()

        for kk in range(ROUNDS):
            cid = s + NS * kk

            @pl.when(cid < NRCHUNK)
            def _():
                r0 = cid * RCHUNK
                pltpu.async_copy(acc.at[pl.ds(r0, RCHUNK)],
                                 acc_out.at[c, pl.ds(r0, RCHUNK)], semz)
                pltpu.async_copy(dacc.at[pl.ds(r0, RCHUNK)], dstage.at[kk],
                                 semz)

        for kk in range(ROUNDS):
            cid = s + NS * kk

            @pl.when(cid < NRCHUNK)
            def _():
                r0 = cid * RCHUNK
                pltpu.make_async_copy(acc.at[pl.ds(r0, RCHUNK)],
                                      acc_out.at[c, pl.ds(r0, RCHUNK)],
                                      semz).wait()
                pltpu.make_async_copy(dacc.at[pl.ds(r0, RCHUNK)],
                                      dstage.at[kk], semz).wait()

                @pl.when(c == 0)
                def _():
                    pltpu.async_copy(dstage.at[kk],
                                     deg0_out.at[pl.ds(r0, RCHUNK)], semz)

                @pl.when(c == 1)
                def _():
                    pltpu.async_copy(dstage.at[kk],
                                     deg1_out.at[pl.ds(r0, RCHUNK)], semz)

        for kk in range(ROUNDS):
            cid = s + NS * kk

            @pl.when(cid < NRCHUNK)
            def _():
                r0 = cid * RCHUNK

                @pl.when(c == 0)
                def _():
                    pltpu.make_async_copy(
                        dstage.at[kk], deg0_out.at[pl.ds(r0, RCHUNK)],
                        semz).wait()

                @pl.when(c == 1)
                def _():
                    pltpu.make_async_copy(
                        dstage.at[kk], deg1_out.at[pl.ds(r0, RCHUNK)],
                        semz).wait()

    return k(ndata, eidx_flat, zrows, zdeg, ones_blk)


ROW_BLK = 1000


def _tc_finish_body(nd_ref, p_ref, d0_ref, d1_ref, w_ref, b_ref, o_ref):
    deg = d0_ref[0] + d1_ref[0]
    dcol = jnp.transpose(deg)
    agg = (p_ref[0] + p_ref[1]) / jnp.maximum(dcol, 1.0)
    rst = nd_ref[...] + agg
    o_ref[...] = (
        lax.dot_general(rst, w_ref[...], (((1,), (1,)), ((), ())),
                        preferred_element_type=jnp.float32)
        + b_ref[...]
    )


def _tc_finish(ndata, p, d0, d1, w, b2):
    grid = (N // ROW_BLK,)
    return pl.pallas_call(
        _tc_finish_body,
        grid=grid,
        in_specs=[
            pl.BlockSpec((ROW_BLK, D), lambda i: (i, 0)),
            pl.BlockSpec((NC, ROW_BLK, D), lambda i: (0, i, 0)),
            pl.BlockSpec((1, 1, ROW_BLK), lambda i: (i, 0, 0)),
            pl.BlockSpec((1, 1, ROW_BLK), lambda i: (i, 0, 0)),
            pl.BlockSpec((D, D), lambda i: (0, 0)),
            pl.BlockSpec((1, D), lambda i: (0, 0)),
        ],
        out_specs=pl.BlockSpec((ROW_BLK, D), lambda i: (i, 0)),
        out_shape=jax.ShapeDtypeStruct((N, D), jnp.float32),
    )(ndata, p, d0, d1, w, b2)


@jax.jit
def kernel(ndata, edge_index, W, b):
    eidx_flat = edge_index.reshape(2 * E)
    zrows = jnp.zeros((RCHUNK, D), jnp.float32)
    zdeg = jnp.zeros((RCHUNK,), jnp.float32)
    ones_blk = jnp.ones((BLK,), jnp.float32)

    acc, deg0, deg1 = _sc_aggregate(ndata, eidx_flat, zrows, zdeg, ones_blk)

    b2 = b.reshape(1, D)
    return _tc_finish(ndata, acc, deg0.reshape(N // ROW_BLK, 1, ROW_BLK),
                      deg1.reshape(N // ROW_BLK, 1, ROW_BLK), W, b2)


# speedup vs baseline: 1.3034x; 1.0250x over previous
import functools

import jax
import jax.numpy as jnp
from jax import lax
from jax.experimental import pallas as pl
from jax.experimental.pallas import tpu as pltpu
from jax.experimental.pallas import tpu_sc as plsc

N = 10000
E = 320000
D = 128

NC = 2
NS = 16
NW = NC * NS

EDGES_PER_WORKER = E // NW
BLK = 40
NBLK = EDGES_PER_WORKER // BLK
RCHUNK = BLK
NRCHUNK = N // RCHUNK
ROUNDS = (NRCHUNK + NS - 1) // NS
RI = 12
RR = 8
RS = 4
GA = 4


def _sc_aggregate(ndata, eidx_flat, zrows, zdeg, ones_blk):
    mesh = plsc.VectorSubcoreMesh(core_axis_name="c", subcore_axis_name="s")

    @functools.partial(
        pl.kernel,
        mesh=mesh,
        out_type=(
            jax.ShapeDtypeStruct((NC, N, D), jnp.float32),
            jax.ShapeDtypeStruct((N,), jnp.float32),
            jax.ShapeDtypeStruct((N,), jnp.float32),
        ),
        scratch_types=[
            pltpu.VMEM((RI, BLK), jnp.int32),
            pltpu.VMEM((RI, BLK), jnp.int32),
            pltpu.VMEM((RR, BLK, D), jnp.float32),
            pltpu.VMEM((ROUNDS, RCHUNK), jnp.float32),
            pltpu.VMEM((BLK,), jnp.float32),
            pltpu.VMEM_SHARED((N, D), jnp.float32),
            pltpu.VMEM_SHARED((N,), jnp.float32),
            pltpu.SemaphoreType.DMA((RI,)),
            pltpu.SemaphoreType.DMA((RR,)),
            pltpu.SemaphoreType.DMA((RS,)),
            pltpu.SemaphoreType.DMA,
        ],
    )
    def k(ndata_hbm, eidx_hbm, zrows_hbm, zdeg_hbm, ones_hbm,
          acc_out, deg0_out, deg1_out, sidx, didx, rows, dstage,
          ones_v, acc, dacc, semi, semg, sems, semz):
        c = lax.axis_index("c")
        s = lax.axis_index("s")

        pltpu.async_copy(zrows_hbm, rows.at[0], semz)
        pltpu.async_copy(zdeg_hbm, dstage.at[0], semz)
        pltpu.async_copy(ones_hbm, ones_v, semz)
        pltpu.make_async_copy(zrows_hbm, rows.at[0], semz).wait()
        pltpu.make_async_copy(zdeg_hbm, dstage.at[0], semz).wait()
        pltpu.make_async_copy(ones_hbm, ones_v, semz).wait()
        for kk in range(ROUNDS):
            cid = s + NS * kk

            @pl.when(cid < NRCHUNK)
            def _():
                r0 = cid * RCHUNK
                pltpu.async_copy(rows.at[0], acc.at[pl.ds(r0, RCHUNK)],
                                 semz)
                pltpu.async_copy(dstage.at[0], dacc.at[pl.ds(r0, RCHUNK)],
                                 semz)

        for kk in range(ROUNDS):
            cid = s + NS * kk

            @pl.when(cid < NRCHUNK)
            def _():
                r0 = cid * RCHUNK
                pltpu.make_async_copy(rows.at[0],
                                      acc.at[pl.ds(r0, RCHUNK)],
                                      semz).wait()
                pltpu.make_async_copy(dstage.at[0],
                                      dacc.at[pl.ds(r0, RCHUNK)],
                                      semz).wait()

        plsc.subcore_barrier()

        chunk_base = (c * NS + s) * EDGES_PER_WORKER

        def fire_idx(blk):
            isl = lax.rem(blk, RI)
            base = chunk_base + blk * BLK
            pltpu.async_copy(eidx_hbm.at[pl.ds(base, BLK)], sidx.at[isl],
                             semi.at[isl])
            pltpu.async_copy(eidx_hbm.at[pl.ds(E + base, BLK)],
                             didx.at[isl], semi.at[isl])

        def wait_idx(blk):
            isl = lax.rem(blk, RI)
            base = chunk_base + blk * BLK
            pltpu.make_async_copy(eidx_hbm.at[pl.ds(base, BLK)],
                                  sidx.at[isl], semi.at[isl]).wait()
            pltpu.make_async_copy(eidx_hbm.at[pl.ds(E + base, BLK)],
                                  didx.at[isl], semi.at[isl]).wait()

        def fire_gather(blk):
            isl = lax.rem(blk, RI)
            rsl = lax.rem(blk, RR)
            pltpu.async_copy(ndata_hbm.at[sidx.at[isl]], rows.at[rsl],
                             semg.at[rsl])

        def wait_gather(blk):
            isl = lax.rem(blk, RI)
            rsl = lax.rem(blk, RR)
            pltpu.make_async_copy(ndata_hbm.at[sidx.at[isl]], rows.at[rsl],
                                  semg.at[rsl]).wait()

        def fire_scat(blk):
            isl = lax.rem(blk, RI)
            rsl = lax.rem(blk, RR)
            ssl = lax.rem(blk, RS)
            pltpu.async_copy(rows.at[rsl], acc.at[didx.at[isl]],
                             sems.at[ssl], add=True)
            pltpu.async_copy(ones_v, dacc.at[didx.at[isl]], sems.at[ssl],
                             add=True)

        def wait_scat(blk):
            isl = lax.rem(blk, RI)
            rsl = lax.rem(blk, RR)
            ssl = lax.rem(blk, RS)
            pltpu.make_async_copy(rows.at[rsl], acc.at[didx.at[isl]],
                                  sems.at[ssl]).wait()
            pltpu.make_async_copy(ones_v, dacc.at[didx.at[isl]],
                                  sems.at[ssl]).wait()

        for pb in range(2 * GA):
            fire_idx(pb)
        for pb in range(GA):
            wait_idx(pb)
            fire_gather(pb)

        def body(i, carry):
            @pl.when(i >= GA)
            def _():
                wait_scat(i - GA)

            @pl.when(i + 2 * GA < NBLK)
            def _():
                fire_idx(i + 2 * GA)

            @pl.when(i + GA < NBLK)
            def _():
                wait_idx(i + GA)
                fire_gather(i + GA)

            wait_gather(i)
            fire_scat(i)
            return carry

        lax.fori_loop(0, NBLK, body, 0)
        for pb in range(GA):
            wait_scat(NBLK - GA + pb)
        plsc.subcore_barrier()

        for kk in range(ROUNDS):
            cid = s + NS * kk

            @pl.when(cid < NRCHUNK)
            def _():
                r0 = cid * RCHUNK
                pltpu.async_copy(acc.at[pl.ds(r0, RCHUNK)],
                                 acc_out.at[c, pl.ds(r0, RCHUNK)], semz)
                pltpu.async_copy(dacc.at[pl.ds(r0, RCHUNK)], dstage.at[kk],
                                 semz)

        for kk in range(ROUNDS):
            cid = s + NS * kk

            @pl.when(cid < NRCHUNK)
            def _():
                r0 = cid * RCHUNK
                pltpu.make_async_copy(acc.at[pl.ds(r0, RCHUNK)],
                                      acc_out.at[c, pl.ds(r0, RCHUNK)],
                                      semz).wait()
                pltpu.make_async_copy(dacc.at[pl.ds(r0, RCHUNK)],
                                      dstage.at[kk], semz).wait()

                @pl.when(c == 0)
                def _():
                    pltpu.async_copy(dstage.at[kk],
                                     deg0_out.at[pl.ds(r0, RCHUNK)], semz)

                @pl.when(c == 1)
                def _():
                    pltpu.async_copy(dstage.at[kk],
                                     deg1_out.at[pl.ds(r0, RCHUNK)], semz)

        for kk in range(ROUNDS):
            cid = s + NS * kk

            @pl.when(cid < NRCHUNK)
            def _():
                r0 = cid * RCHUNK

                @pl.when(c == 0)
                def _():
                    pltpu.make_async_copy(
                        dstage.at[kk], deg0_out.at[pl.ds(r0, RCHUNK)],
                        semz).wait()

                @pl.when(c == 1)
                def _():
                    pltpu.make_async_copy(
                        dstage.at[kk], deg1_out.at[pl.ds(r0, RCHUNK)],
                        semz).wait()

    return k(ndata, eidx_flat, zrows, zdeg, ones_blk)


ROW_BLK = 1000


def _tc_finish_body(nd_ref, p_ref, d0_ref, d1_ref, w_ref, b_ref, o_ref):
    deg = d0_ref[0] + d1_ref[0]
    dcol = jnp.transpose(deg)
    agg = (p_ref[0] + p_ref[1]) / jnp.maximum(dcol, 1.0)
    rst = nd_ref[...] + agg
    o_ref[...] = (
        lax.dot_general(rst, w_ref[...], (((1,), (1,)), ((), ())),
                        preferred_element_type=jnp.float32)
        + b_ref[...]
    )


def _tc_finish(ndata, p, d0, d1, w, b2):
    grid = (N // ROW_BLK,)
    return pl.pallas_call(
        _tc_finish_body,
        grid=grid,
        in_specs=[
            pl.BlockSpec((ROW_BLK, D), lambda i: (i, 0)),
            pl.BlockSpec((NC, ROW_BLK, D), lambda i: (0, i, 0)),
            pl.BlockSpec((1, 1, ROW_BLK), lambda i: (i, 0, 0)),
            pl.BlockSpec((1, 1, ROW_BLK), lambda i: (i, 0, 0)),
            pl.BlockSpec((D, D), lambda i: (0, 0)),
            pl.BlockSpec((1, D), lambda i: (0, 0)),
        ],
        out_specs=pl.BlockSpec((ROW_BLK, D), lambda i: (i, 0)),
        out_shape=jax.ShapeDtypeStruct((N, D), jnp.float32),
    )(ndata, p, d0, d1, w, b2)


@jax.jit
def kernel(ndata, edge_index, W, b):
    eidx_flat = edge_index.reshape(2 * E)
    zrows = jnp.zeros((RCHUNK, D), jnp.float32)
    zdeg = jnp.zeros((RCHUNK,), jnp.float32)
    ones_blk = jnp.ones((BLK,), jnp.float32)

    acc, deg0, deg1 = _sc_aggregate(ndata, eidx_flat, zrows, zdeg, ones_blk)

    b2 = b.reshape(1, D)
    return _tc_finish(ndata, acc, deg0.reshape(N // ROW_BLK, 1, ROW_BLK),
                      deg1.reshape(N // ROW_BLK, 1, ROW_BLK), W, b2)


# speedup vs baseline: 1.3247x; 1.0163x over previous
import functools

import jax
import jax.numpy as jnp
from jax import lax
from jax.experimental import pallas as pl
from jax.experimental.pallas import tpu as pltpu
from jax.experimental.pallas import tpu_sc as plsc

N = 10000
E = 320000
D = 128

NC = 2
NS = 16
NW = NC * NS

EDGES_PER_WORKER = E // NW
BLK = 40
NBLK = EDGES_PER_WORKER // BLK
RCHUNK = BLK
NRCHUNK = N // RCHUNK
ROUNDS = (NRCHUNK + NS - 1) // NS
RI = 12
RR = 8
RS = 4
GA = 4


def _sc_aggregate(ndata, eidx_flat, zrows, zdeg, ones_blk):
    mesh = plsc.VectorSubcoreMesh(core_axis_name="c", subcore_axis_name="s")

    @functools.partial(
        pl.kernel,
        mesh=mesh,
        out_type=(
            jax.ShapeDtypeStruct((NC, N, D), jnp.float32),
            jax.ShapeDtypeStruct((N,), jnp.float32),
            jax.ShapeDtypeStruct((N,), jnp.float32),
        ),
        scratch_types=[
            pltpu.VMEM((RI, BLK), jnp.int32),
            pltpu.VMEM((RI, BLK), jnp.int32),
            pltpu.VMEM((RR, BLK, D), jnp.float32),
            pltpu.VMEM((ROUNDS, RCHUNK), jnp.float32),
            pltpu.VMEM((BLK,), jnp.float32),
            pltpu.VMEM_SHARED((N, D), jnp.float32),
            pltpu.VMEM_SHARED((N,), jnp.float32),
            pltpu.SemaphoreType.DMA((RI,)),
            pltpu.SemaphoreType.DMA((RR,)),
            pltpu.SemaphoreType.DMA((RS,)),
            pltpu.SemaphoreType.DMA,
        ],
    )
    def k(ndata_hbm, eidx_hbm, zrows_hbm, zdeg_hbm, ones_hbm,
          acc_out, deg0_out, deg1_out, sidx, didx, rows, dstage,
          ones_v, acc, dacc, semi, semg, sems, semz):
        c = lax.axis_index("c")
        s = lax.axis_index("s")

        pltpu.async_copy(zrows_hbm, rows.at[0], semz)
        pltpu.async_copy(zdeg_hbm, dstage.at[0], semz)
        pltpu.async_copy(ones_hbm, ones_v, semz)
        pltpu.make_async_copy(zrows_hbm, rows.at[0], semz).wait()
        pltpu.make_async_copy(zdeg_hbm, dstage.at[0], semz).wait()
        pltpu.make_async_copy(ones_hbm, ones_v, semz).wait()
        for kk in range(ROUNDS):
            cid = s + NS * kk

            @pl.when(cid < NRCHUNK)
            def _():
                r0 = cid * RCHUNK
                pltpu.async_copy(rows.at[0], acc.at[pl.ds(r0, RCHUNK)],
                                 semz)
                pltpu.async_copy(dstage.at[0], dacc.at[pl.ds(r0, RCHUNK)],
                                 semz)

        for kk in range(ROUNDS):
            cid = s + NS * kk

            @pl.when(cid < NRCHUNK)
            def _():
                r0 = cid * RCHUNK
                pltpu.make_async_copy(rows.at[0],
                                      acc.at[pl.ds(r0, RCHUNK)],
                                      semz).wait()
                pltpu.make_async_copy(dstage.at[0],
                                      dacc.at[pl.ds(r0, RCHUNK)],
                                      semz).wait()

        plsc.subcore_barrier()

        chunk_base = (c * NS + s) * EDGES_PER_WORKER

        def fire_idx(blk):
            isl = lax.rem(blk, RI)
            base = chunk_base + blk * BLK
            pltpu.async_copy(eidx_hbm.at[pl.ds(base, BLK)], sidx.at[isl],
                             semi.at[isl])
            pltpu.async_copy(eidx_hbm.at[pl.ds(E + base, BLK)],
                             didx.at[isl], semi.at[isl])

        def wait_idx(blk):
            isl = lax.rem(blk, RI)
            base = chunk_base + blk * BLK
            pltpu.make_async_copy(eidx_hbm.at[pl.ds(base, BLK)],
                                  sidx.at[isl], semi.at[isl]).wait()
            pltpu.make_async_copy(eidx_hbm.at[pl.ds(E + base, BLK)],
                                  didx.at[isl], semi.at[isl]).wait()

        def fire_gather(blk):
            isl = lax.rem(blk, RI)
            rsl = lax.rem(blk, RR)
            pltpu.async_copy(ndata_hbm.at[sidx.at[isl]], rows.at[rsl],
                             semg.at[rsl])

        def wait_gather(blk):
            isl = lax.rem(blk, RI)
            rsl = lax.rem(blk, RR)
            pltpu.make_async_copy(ndata_hbm.at[sidx.at[isl]], rows.at[rsl],
                                  semg.at[rsl]).wait()

        def fire_scat(blk):
            isl = lax.rem(blk, RI)
            rsl = lax.rem(blk, RR)
            ssl = lax.rem(blk, RS)
            pltpu.async_copy(rows.at[rsl], acc.at[didx.at[isl]],
                             sems.at[ssl], add=True)
            pltpu.async_copy(ones_v, dacc.at[didx.at[isl]], sems.at[ssl],
                             add=True)

        def wait_scat(blk):
            isl = lax.rem(blk, RI)
            rsl = lax.rem(blk, RR)
            ssl = lax.rem(blk, RS)
            pltpu.make_async_copy(rows.at[rsl], acc.at[didx.at[isl]],
                                  sems.at[ssl]).wait()
            pltpu.make_async_copy(ones_v, dacc.at[didx.at[isl]],
                                  sems.at[ssl]).wait()

        for pb in range(2 * GA):
            fire_idx(pb)
        for pb in range(GA):
            wait_idx(pb)
            fire_gather(pb)

        def body(i, carry):
            @pl.when(i >= GA)
            def _():
                wait_scat(i - GA)

            @pl.when(i + 2 * GA < NBLK)
            def _():
                fire_idx(i + 2 * GA)

            @pl.when(i + GA < NBLK)
            def _():
                wait_idx(i + GA)
                fire_gather(i + GA)

            wait_gather(i)
            fire_scat(i)
            return carry

        lax.fori_loop(0, NBLK, body, 0)
        for pb in range(GA):
            wait_scat(NBLK - GA + pb)
        plsc.subcore_barrier()

        for kk in range(ROUNDS):
            cid = s + NS * kk

            @pl.when(cid < NRCHUNK)
            def _():
                r0 = cid * RCHUNK
                pltpu.async_copy(acc.at[pl.ds(r0, RCHUNK)],
                                 acc_out.at[c, pl.ds(r0, RCHUNK)], semz)
                pltpu.async_copy(dacc.at[pl.ds(r0, RCHUNK)], dstage.at[kk],
                                 semz)

        for kk in range(ROUNDS):
            cid = s + NS * kk

            @pl.when(cid < NRCHUNK)
            def _():
                r0 = cid * RCHUNK
                pltpu.make_async_copy(acc.at[pl.ds(r0, RCHUNK)],
                                      acc_out.at[c, pl.ds(r0, RCHUNK)],
                                      semz).wait()
                pltpu.make_async_copy(dacc.at[pl.ds(r0, RCHUNK)],
                                      dstage.at[kk], semz).wait()

                @pl.when(c == 0)
                def _():
                    pltpu.async_copy(dstage.at[kk],
                                     deg0_out.at[pl.ds(r0, RCHUNK)], semz)

                @pl.when(c == 1)
                def _():
                    pltpu.async_copy(dstage.at[kk],
                                     deg1_out.at[pl.ds(r0, RCHUNK)], semz)

        for kk in range(ROUNDS):
            cid = s + NS * kk

            @pl.when(cid < NRCHUNK)
            def _():
                r0 = cid * RCHUNK

                @pl.when(c == 0)
                def _():
                    pltpu.make_async_copy(
                        dstage.at[kk], deg0_out.at[pl.ds(r0, RCHUNK)],
                        semz).wait()

                @pl.when(c == 1)
                def _():
                    pltpu.make_async_copy(
                        dstage.at[kk], deg1_out.at[pl.ds(r0, RCHUNK)],
                        semz).wait()

    return k(ndata, eidx_flat, zrows, zdeg, ones_blk)


ROW_BLK = 2000


def _tc_finish_body(nd_ref, p_ref, d0_ref, d1_ref, w_ref, b_ref, o_ref):
    deg = d0_ref[0] + d1_ref[0]
    dcol = jnp.transpose(deg)
    agg = (p_ref[0] + p_ref[1]) / jnp.maximum(dcol, 1.0)
    rst = nd_ref[...] + agg
    o_ref[...] = (
        lax.dot_general(rst, w_ref[...], (((1,), (1,)), ((), ())),
                        preferred_element_type=jnp.float32)
        + b_ref[...]
    )


def _tc_finish(ndata, p, d0, d1, w, b2):
    grid = (N // ROW_BLK,)
    return pl.pallas_call(
        _tc_finish_body,
        grid=grid,
        in_specs=[
            pl.BlockSpec((ROW_BLK, D), lambda i: (i, 0)),
            pl.BlockSpec((NC, ROW_BLK, D), lambda i: (0, i, 0)),
            pl.BlockSpec((1, 1, ROW_BLK), lambda i: (i, 0, 0)),
            pl.BlockSpec((1, 1, ROW_BLK), lambda i: (i, 0, 0)),
            pl.BlockSpec((D, D), lambda i: (0, 0)),
            pl.BlockSpec((1, D), lambda i: (0, 0)),
        ],
        out_specs=pl.BlockSpec((ROW_BLK, D), lambda i: (i, 0)),
        out_shape=jax.ShapeDtypeStruct((N, D), jnp.float32),
    )(ndata, p, d0, d1, w, b2)


@jax.jit
def kernel(ndata, edge_index, W, b):
    eidx_flat = edge_index.reshape(2 * E)
    zrows = jnp.zeros((RCHUNK, D), jnp.float32)
    zdeg = jnp.zeros((RCHUNK,), jnp.float32)
    ones_blk = jnp.ones((BLK,), jnp.float32)

    acc, deg0, deg1 = _sc_aggregate(ndata, eidx_flat, zrows, zdeg, ones_blk)

    b2 = b.reshape(1, D)
    return _tc_finish(ndata, acc, deg0.reshape(N // ROW_BLK, 1, ROW_BLK),
                      deg1.reshape(N // ROW_BLK, 1, ROW_BLK), W, b2)
